# baseline (speedup 1.0000x reference)
import functools

import jax
import jax.numpy as jnp
from jax import lax
from jax.experimental import pallas as pl
from jax.experimental.pallas import tpu as pltpu
from jax.experimental.pallas import tpu_sc as plsc


@functools.cache
def _build(B, F, V, D):
    info = plsc.get_sparse_core_info()
    NC, NS, L = info.num_cores, info.num_subcores, info.num_lanes
    NW = NC * NS
    BF = B * F
    assert BF % NW == 0
    b_per_w = BF // NW
    CH = 32 * F
    assert b_per_w % CH == 0
    n_chunks = b_per_w // CH
    mesh = plsc.VectorSubcoreMesh(core_axis_name="c", subcore_axis_name="s")

    @functools.partial(
        pl.kernel,
        mesh=mesh,
        out_type=jax.ShapeDtypeStruct((BF, D), jnp.float32),
        scratch_types=[
            pltpu.VMEM((CH,), jnp.int32),
            pltpu.VMEM((CH, D), jnp.float32),
            pltpu.VMEM((F, D), jnp.float32),
            pltpu.SemaphoreType.DMA,
        ],
        compiler_params=pltpu.CompilerParams(use_tc_tiling_on_sc=False),
    )
    def gather_bias(table_hbm, idx_hbm, col_hbm, out_hbm, idx_v, rows_v, bias_v, sem):
        wid = lax.axis_index("s") * NC + lax.axis_index("c")
        base = wid * b_per_w
        pltpu.sync_copy(col_hbm, bias_v)
        for c in range(n_chunks):
            off = base + c * CH
            pltpu.sync_copy(idx_hbm.at[pl.ds(off, CH)], idx_v)
            pltpu.async_copy(table_hbm.at[idx_v], rows_v, sem).wait()

            def g_body(g, carry):
                for jj in range(F):
                    r = g * F + jj
                    for h in range(0, D, L):
                        rows_v[r, pl.ds(h, L)] = (
                            rows_v[r, pl.ds(h, L)] + bias_v[jj, pl.ds(h, L)]
                        )
                return carry

            lax.fori_loop(0, CH // F, g_body, 0)
            pltpu.sync_copy(rows_v, out_hbm.at[pl.ds(off, CH)])

    return gather_bias


def kernel(x_cat, tables, col_embed):
    F, V, D = tables.shape
    B = x_cat.shape[0]
    flat_idx = (
        x_cat.astype(jnp.int32) + (jnp.arange(F, dtype=jnp.int32) * V)[None, :]
    ).reshape(-1)
    table_flat = tables.reshape(F * V, D)
    out_flat = _build(B, F, V, D)(table_flat, flat_idx, col_embed)
    return out_flat.reshape(B, F, D)


# speedup vs baseline: 3.7384x; 3.7384x over previous
import functools

import jax
import jax.numpy as jnp
from jax import lax
from jax.experimental import pallas as pl
from jax.experimental.pallas import tpu as pltpu
from jax.experimental.pallas import tpu_sc as plsc


@functools.cache
def _build(B, F, V, D):
    info = plsc.get_sparse_core_info()
    NC, NS, L = info.num_cores, info.num_subcores, info.num_lanes
    NW = NC * NS
    assert D == NW, "one emb_dim plane per vector subcore"
    CHO = 4096
    assert B % CHO == 0 and CHO % L == 0
    mesh = plsc.VectorSubcoreMesh(core_axis_name="c", subcore_axis_name="s")

    @functools.partial(
        pl.kernel,
        mesh=mesh,
        out_type=jax.ShapeDtypeStruct((F, D, B), jnp.float32),
        scratch_types=[
            pltpu.VMEM((V,), jnp.float32),
            pltpu.VMEM((CHO,), jnp.int32),
            pltpu.VMEM((CHO,), jnp.float32),
            pltpu.VMEM((D,), jnp.float32),
        ],
        compiler_params=pltpu.CompilerParams(needs_layout_passes=False),
    )
    def gather_bias(tab_t, x_t, col, out_t, plane_v, idx_v, out_v, col_v):
        w = lax.axis_index("s") * NC + lax.axis_index("c")
        w16 = jnp.full((L,), w, jnp.int32)
        for f in range(F):
            pltpu.sync_copy(tab_t.at[f, w], plane_v)
            pltpu.sync_copy(col.at[f], col_v)
            bias = plsc.load_gather(col_v, [w16])
            for c in range(B // CHO):
                pltpu.sync_copy(x_t.at[f, pl.ds(c * CHO, CHO)], idx_v)

                def ibody(i, carry):
                    idx16 = idx_v[pl.ds(i * L, L)]
                    out_v[pl.ds(i * L, L)] = (
                        plsc.load_gather(plane_v, [idx16]) + bias
                    )
                    return carry

                lax.fori_loop(0, CHO // L, ibody, 0)
                pltpu.sync_copy(out_v, out_t.at[f, w, pl.ds(c * CHO, CHO)])

    return gather_bias


def kernel(x_cat, tables, col_embed):
    F, V, D = tables.shape
    B = x_cat.shape[0]
    tab_t = tables.transpose(0, 2, 1)
    x_t = x_cat.astype(jnp.int32).T
    out_t = _build(B, F, V, D)(tab_t, x_t, col_embed)
    return out_t.transpose(2, 0, 1)


# speedup vs baseline: 4.8982x; 1.3102x over previous
import functools

import jax
import jax.numpy as jnp
from jax import lax
from jax.experimental import pallas as pl
from jax.experimental.pallas import tpu as pltpu
from jax.experimental.pallas import tpu_sc as plsc


@functools.cache
def _build(B, F, V, D):
    info = plsc.get_sparse_core_info()
    NC, NS, L = info.num_cores, info.num_subcores, info.num_lanes
    NW = NC * NS
    assert D == NW, "one emb_dim plane per vector subcore"
    CHO = 2048
    NCH = B // CHO
    assert B % CHO == 0 and CHO % L == 0
    mesh = plsc.VectorSubcoreMesh(core_axis_name="c", subcore_axis_name="s")

    @functools.partial(
        pl.kernel,
        mesh=mesh,
        out_type=jax.ShapeDtypeStruct((F, D, B), jnp.float32),
        scratch_types=[
            pltpu.VMEM((V,), jnp.float32),
            pltpu.VMEM((2, CHO), jnp.int32),
            pltpu.VMEM((2, CHO), jnp.float32),
            pltpu.VMEM((D,), jnp.float32),
            pltpu.SemaphoreType.DMA,
            pltpu.SemaphoreType.DMA,
            pltpu.SemaphoreType.DMA,
            pltpu.SemaphoreType.DMA,
        ],
        compiler_params=pltpu.CompilerParams(needs_layout_passes=False),
    )
    def gather_bias(tab_t, x_t, col, out_t, plane_v, idx_v, out_v, col_v,
                    sem_i0, sem_i1, sem_o0, sem_o1):
        w = lax.axis_index("s") * NC + lax.axis_index("c")
        w16 = jnp.full((L,), w, jnp.int32)
        sem_i = (sem_i0, sem_i1)
        sem_o = (sem_o0, sem_o1)

        def field_body(f, carry):
            pltpu.sync_copy(tab_t.at[f, w], plane_v)
            pltpu.sync_copy(col.at[f], col_v)
            bias = plsc.load_gather(col_v, [w16])
            idx_cp = [None, None]
            out_cp = [None, None]
            idx_cp[0] = pltpu.async_copy(
                x_t.at[f, pl.ds(0, CHO)], idx_v.at[0], sem_i[0])
            for c in range(NCH):
                b0 = c % 2
                idx_cp[b0].wait()
                if c + 1 < NCH:
                    b1 = (c + 1) % 2
                    idx_cp[b1] = pltpu.async_copy(
                        x_t.at[f, pl.ds((c + 1) * CHO, CHO)],
                        idx_v.at[b1], sem_i[b1])
                if out_cp[b0] is not None:
                    out_cp[b0].wait()

                @plsc.parallel_loop(0, CHO // L, unroll=8)
                def ibody(i):
                    idx16 = idx_v[b0, pl.ds(i * L, L)]
                    out_v[b0, pl.ds(i * L, L)] = (
                        plsc.load_gather(plane_v, [idx16]) + bias
                    )

                out_cp[b0] = pltpu.async_copy(
                    out_v.at[b0], out_t.at[f, w, pl.ds(c * CHO, CHO)],
                    sem_o[b0])
            out_cp[0].wait()
            out_cp[1].wait()
            return carry

        lax.fori_loop(0, F, field_body, 0)

    return gather_bias


def kernel(x_cat, tables, col_embed):
    F, V, D = tables.shape
    B = x_cat.shape[0]
    tab_t = tables.transpose(0, 2, 1)
    x_t = x_cat.astype(jnp.int32).T
    out_t = _build(B, F, V, D)(tab_t, x_t, col_embed)
    return out_t.transpose(2, 0, 1)
